# baseline (speedup 1.0000x reference)
import functools

import jax
import jax.numpy as jnp
from jax import lax
from jax.experimental import pallas as pl
from jax.experimental.pallas import tpu as pltpu
from jax.experimental.pallas import tpu_sc as plsc

_MAXT = 1.0
_ROWS = 50000
_DIM = 128
_DELTAT = _MAXT / _ROWS
_LANES = 16
_SUB = 80
_NBUF = 2


@functools.cache
def _sc_gather(R, C, NC, NS):
    NW = NC * NS
    b_per_w = R // NW
    n_chunks = b_per_w // C
    n_sub = C // _SUB
    assert n_chunks % _NBUF == 0
    mesh = plsc.VectorSubcoreMesh(core_axis_name="c", subcore_axis_name="s")

    buf_types = [pltpu.VMEM((b_per_w,), jnp.float32)]
    for _ in range(_NBUF):
        buf_types += [
            pltpu.VMEM((C,), jnp.int32),
            pltpu.VMEM((C, _DIM), jnp.float32),
            pltpu.SemaphoreType.DMA,
            pltpu.SemaphoreType.DMA,
        ]

    @functools.partial(
        pl.kernel,
        out_type=jax.ShapeDtypeStruct((R, _DIM), jnp.float32),
        mesh=mesh,
        scratch_types=buf_types,
    )
    def k(ts_hbm, table_hbm, out_hbm, ts_all, *bufs):
        wid = lax.axis_index("s") * NC + lax.axis_index("c")
        base = wid * b_per_w
        idx_v = [bufs[4 * b + 0] for b in range(_NBUF)]
        rows_v = [bufs[4 * b + 1] for b in range(_NBUF)]
        gsem = [bufs[4 * b + 2] for b in range(_NBUF)]
        osem = [bufs[4 * b + 3] for b in range(_NBUF)]

        pltpu.sync_copy(ts_hbm.at[pl.ds(base, b_per_w)], ts_all)

        def stage(chunk, b):
            def idx_body(i, c):
                v = ts_all[pl.ds(chunk * C + i * _LANES, _LANES)]
                q = (v / _DELTAT).astype(jnp.int32)
                q = jnp.minimum(jnp.maximum(q, 0), _ROWS - 1)
                idx_v[b][pl.ds(i * _LANES, _LANES)] = q
                return c

            lax.fori_loop(0, C // _LANES, idx_body, 0)
            for j in range(n_sub):
                pltpu.async_copy(
                    table_hbm.at[idx_v[b].at[pl.ds(j * _SUB, _SUB)]],
                    rows_v[b].at[pl.ds(j * _SUB, _SUB)],
                    gsem[b],
                )

        def wait_gather(b):
            for j in range(n_sub):
                pltpu.make_async_copy(
                    table_hbm.at[idx_v[b].at[pl.ds(j * _SUB, _SUB)]],
                    rows_v[b].at[pl.ds(j * _SUB, _SUB)],
                    gsem[b],
                ).wait()

        def fire_scatter(chunk, b):
            start = base + chunk * C
            pltpu.async_copy(rows_v[b], out_hbm.at[pl.ds(start, C)], osem[b])

        def wait_scatter(chunk, b):
            start = base + chunk * C
            pltpu.make_async_copy(
                rows_v[b], out_hbm.at[pl.ds(start, C)], osem[b]
            ).wait()

        for b in range(_NBUF):
            stage(b, b)

        def pair_body(g, carry):
            for b in range(_NBUF):
                chunk = g * _NBUF + b
                wait_gather(b)
                fire_scatter(chunk, b)
                nxt = chunk + _NBUF
                def idx_body(i, c, b=b, nxt=nxt):
                    v = ts_all[pl.ds(nxt * C + i * _LANES, _LANES)]
                    q = (v / _DELTAT).astype(jnp.int32)
                    q = jnp.minimum(jnp.maximum(q, 0), _ROWS - 1)
                    idx_v[b][pl.ds(i * _LANES, _LANES)] = q
                    return c

                lax.fori_loop(0, C // _LANES, idx_body, 0)
                wait_scatter(chunk, b)
                for j in range(n_sub):
                    pltpu.async_copy(
                        table_hbm.at[idx_v[b].at[pl.ds(j * _SUB, _SUB)]],
                        rows_v[b].at[pl.ds(j * _SUB, _SUB)],
                        gsem[b],
                    )
            return carry

        lax.fori_loop(0, n_chunks // _NBUF - 1, pair_body, 0)

        for b in range(_NBUF):
            chunk = n_chunks - _NBUF + b
            wait_gather(b)
            fire_scatter(chunk, b)
        for b in range(_NBUF):
            wait_scatter(n_chunks - _NBUF + b, b)

    return k


def kernel(timestamps, table):
    B, T = timestamps.shape
    R = B * T
    info = plsc.get_sparse_core_info()
    k = _sc_gather(R, 400, info.num_cores, info.num_subcores)
    out = k(jnp.reshape(timestamps, (R,)), table)
    return jnp.reshape(out, (B, T, _DIM))


# speedup vs baseline: 1.0215x; 1.0215x over previous
import functools

import jax
import jax.numpy as jnp
from jax import lax
from jax.experimental import pallas as pl
from jax.experimental.pallas import tpu as pltpu
from jax.experimental.pallas import tpu_sc as plsc

_MAXT = 1.0
_ROWS = 50000
_DIM = 128
_DELTAT = _MAXT / _ROWS
_LANES = 16
_SUB = 128
_NBUF = 2
_NSLAB = 4


@functools.cache
def _sc_gather(R, C, NC, NS):
    NW = NC * NS
    b_per_w = R // NW
    n_chunks = b_per_w // C
    n_sub = C // _SUB
    assert n_chunks % _NSLAB == 0 and _NSLAB % _NBUF == 0
    mesh = plsc.VectorSubcoreMesh(core_axis_name="c", subcore_axis_name="s")

    buf_types = [pltpu.VMEM((b_per_w,), jnp.float32)]
    for _ in range(_NBUF):
        buf_types += [
            pltpu.VMEM((C,), jnp.int32),
            pltpu.VMEM((C, _DIM), jnp.float32),
            pltpu.SemaphoreType.DMA,
            pltpu.SemaphoreType.DMA,
        ]
    buf_types += [pltpu.VMEM_SHARED((NS, _NSLAB, C, _DIM), jnp.float32)]
    buf_types += [pltpu.SemaphoreType.DMA] * _NSLAB

    @functools.partial(
        pl.kernel,
        out_type=jax.ShapeDtypeStruct((R, _DIM), jnp.float32),
        mesh=mesh,
        scratch_types=buf_types,
    )
    def k(ts_hbm, table_hbm, out_hbm, ts_all, *bufs):
        sid = lax.axis_index("s")
        wid = sid * NC + lax.axis_index("c")
        base = wid * b_per_w
        idx_v = [bufs[4 * b + 0] for b in range(_NBUF)]
        rows_v = [bufs[4 * b + 1] for b in range(_NBUF)]
        gsem = [bufs[4 * b + 2] for b in range(_NBUF)]
        hsem = [bufs[4 * b + 3] for b in range(_NBUF)]
        shared = bufs[4 * _NBUF]
        osem = list(bufs[4 * _NBUF + 1:])

        pltpu.sync_copy(ts_hbm.at[pl.ds(base, b_per_w)], ts_all)

        def idx_compute(chunk, b):
            def idx_body(i, c):
                v = ts_all[pl.ds(chunk * C + i * _LANES, _LANES)]
                q = (v / _DELTAT).astype(jnp.int32)
                q = jnp.minimum(jnp.maximum(q, 0), _ROWS - 1)
                idx_v[b][pl.ds(i * _LANES, _LANES)] = q
                return c

            lax.fori_loop(0, C // _LANES, idx_body, 0)

        def fire_gather(b):
            for j in range(n_sub):
                pltpu.async_copy(
                    table_hbm.at[idx_v[b].at[pl.ds(j * _SUB, _SUB)]],
                    rows_v[b].at[pl.ds(j * _SUB, _SUB)],
                    gsem[b],
                )

        def wait_gather(b):
            for j in range(n_sub):
                pltpu.make_async_copy(
                    table_hbm.at[idx_v[b].at[pl.ds(j * _SUB, _SUB)]],
                    rows_v[b].at[pl.ds(j * _SUB, _SUB)],
                    gsem[b],
                ).wait()

        def wait_drain(s):
            pltpu.make_async_copy(
                shared.at[sid, s], out_hbm.at[pl.ds(base, C)], osem[s]
            ).wait()

        def body(chunk, b, s, first, last):
            wait_gather(b)
            if not first:
                wait_drain(s)
            pltpu.async_copy(rows_v[b], shared.at[sid, s], hsem[b])
            if not last:
                idx_compute(chunk + _NBUF, b)
            pltpu.make_async_copy(rows_v[b], shared.at[sid, s], hsem[b]).wait()
            start = base + chunk * C
            pltpu.async_copy(shared.at[sid, s], out_hbm.at[pl.ds(start, C)], osem[s])
            if not last:
                fire_gather(b)

        for b in range(_NBUF):
            idx_compute(b, b)
            fire_gather(b)

        for c in range(_NSLAB):
            body(c, c % _NBUF, c, True, False)

        def group_body(g, carry):
            for i in range(_NSLAB):
                chunk = g * _NSLAB + i
                body(chunk, i % _NBUF, i, False, False)
            return carry

        lax.fori_loop(1, n_chunks // _NSLAB - 1, group_body, 0)

        for i in range(_NSLAB):
            body(n_chunks - _NSLAB + i, i % _NBUF, i, False,
                 i >= _NSLAB - _NBUF)
        for s in range(_NSLAB):
            wait_drain(s)

    return k


def kernel(timestamps, table):
    B, T = timestamps.shape
    R = B * T
    info = plsc.get_sparse_core_info()
    k = _sc_gather(R, 128, info.num_cores, info.num_subcores)
    out = k(jnp.reshape(timestamps, (R,)), table)
    return jnp.reshape(out, (B, T, _DIM))
